# baseline (speedup 1.0000x reference)
import jax
import jax.numpy as jnp
from jax.experimental import pallas as pl
from jax.experimental.pallas import tpu as pltpu

_IN_F = 128
_OUT_F = 128


def _fused_affine_kernel(w_ref, x0_ref, g0_ref, x1_ref, y0_ref, yg_ref, y1_ref):
    w0 = w_ref[0, :_IN_F, :].astype(jnp.bfloat16)
    b0 = w_ref[0, _IN_F:_IN_F + 1, :]
    w1 = w_ref[1, :_IN_F, :].astype(jnp.bfloat16)
    b1 = w_ref[1, _IN_F:_IN_F + 1, :]
    y0_ref[...] = jnp.dot(x0_ref[...].astype(jnp.bfloat16), w0,
                          preferred_element_type=jnp.float32) + b0
    yg_ref[...] = jnp.dot(g0_ref[...].astype(jnp.bfloat16), w0,
                          preferred_element_type=jnp.float32) + b0
    y1_ref[...] = jnp.dot(x1_ref[...].astype(jnp.bfloat16), w1,
                          preferred_element_type=jnp.float32) + b1


def _affine_kernel(x_ref, w_ref, o_ref):
    w = w_ref[0, :_IN_F, :]
    b = w_ref[0, _IN_F:_IN_F + 1, :]
    o_ref[...] = jnp.dot(x_ref[...], w, preferred_element_type=jnp.float32) + b


def _single_affine(x2d, w_fold):
    m = x2d.shape[0]
    tm = 1024
    while tm > 8 and m % tm:
        tm //= 2
    grid = pl.cdiv(m, tm)
    return pl.pallas_call(
        _affine_kernel,
        grid=(grid,),
        in_specs=[
            pl.BlockSpec((tm, _IN_F), lambda i: (i, 0)),
            pl.BlockSpec((1, w_fold.shape[1], _OUT_F), lambda i: (0, 0, 0)),
        ],
        out_specs=pl.BlockSpec((tm, _OUT_F), lambda i: (i, 0)),
        out_shape=jax.ShapeDtypeStruct((m, _OUT_F), x2d.dtype),
        compiler_params=pltpu.CompilerParams(
            dimension_semantics=("parallel",),
            vmem_limit_bytes=64 << 20,
        ),
    )(x2d, w_fold)


def _fused_affine(x0, g0, x1, w_folded, grid):
    m0, mg, m1 = x0.shape[0], g0.shape[0], x1.shape[0]
    bm0, bmg, bm1 = m0 // grid, mg // grid, m1 // grid
    k_rows = w_folded.shape[1]
    return pl.pallas_call(
        _fused_affine_kernel,
        grid=(grid,),
        in_specs=[
            pl.BlockSpec((2, k_rows, _OUT_F), lambda i: (0, 0, 0)),
            pl.BlockSpec((bm0, _IN_F), lambda i: (i, 0)),
            pl.BlockSpec((bmg, _IN_F), lambda i: (i, 0)),
            pl.BlockSpec((bm1, _IN_F), lambda i: (i, 0)),
        ],
        out_specs=[
            pl.BlockSpec((bm0, _OUT_F), lambda i: (i, 0)),
            pl.BlockSpec((bmg, _OUT_F), lambda i: (i, 0)),
            pl.BlockSpec((bm1, _OUT_F), lambda i: (i, 0)),
        ],
        out_shape=[
            jax.ShapeDtypeStruct((m0, _OUT_F), x0.dtype),
            jax.ShapeDtypeStruct((mg, _OUT_F), g0.dtype),
            jax.ShapeDtypeStruct((m1, _OUT_F), x1.dtype),
        ],
        compiler_params=pltpu.CompilerParams(
            dimension_semantics=("parallel",),
            vmem_limit_bytes=64 << 20,
        ),
    )(w_folded, x0, g0, x1)


def kernel(key0_values, key0_grad_positions, key1_values, w_folded, w_block_diag):
    del w_block_diag
    x0 = jnp.asarray(key0_values, jnp.float32).reshape(-1, _IN_F)
    g0 = jnp.asarray(key0_grad_positions, jnp.float32).reshape(-1, _IN_F)
    x1 = jnp.asarray(key1_values, jnp.float32).reshape(-1, _IN_F)

    grid = 0
    for g in (2,):
        if all(m % g == 0 and (m // g) % 8 == 0
               for m in (x0.shape[0], g0.shape[0], x1.shape[0])):
            grid = g
            break
    if grid:
        y0, yg, y1 = _fused_affine(x0, g0, x1, w_folded, grid)
    else:
        y0 = _single_affine(x0, w_folded[0:1])
        yg = _single_affine(g0, w_folded[0:1])
        y1 = _single_affine(x1, w_folded[1:2])

    return {
        "key_0": {
            "values": y0.reshape(*key0_values.shape[:-1], _OUT_F),
            "gradients": {
                "positions": yg.reshape(*key0_grad_positions.shape[:-1], _OUT_F),
            },
        },
        "key_1": {
            "values": y1.reshape(*key1_values.shape[:-1], _OUT_F),
            "gradients": {},
        },
    }


# speedup vs baseline: 1.0356x; 1.0356x over previous
import jax
import jax.numpy as jnp
from jax.experimental import pallas as pl
from jax.experimental.pallas import tpu as pltpu

_IN_F = 128
_OUT_F = 128


def _fused_affine_kernel(w_ref, x0_ref, g0_ref, x1_ref, y0_ref, yg_ref, y1_ref):
    w0 = w_ref[0, :_IN_F, :]
    b0 = w_ref[0, _IN_F:_IN_F + 1, :]
    w1 = w_ref[1, :_IN_F, :]
    b1 = w_ref[1, _IN_F:_IN_F + 1, :]
    y0_ref[...] = jnp.dot(x0_ref[...], w0, preferred_element_type=jnp.float32) + b0
    yg_ref[...] = jnp.dot(g0_ref[...], w0, preferred_element_type=jnp.float32) + b0
    y1_ref[...] = jnp.dot(x1_ref[...], w1, preferred_element_type=jnp.float32) + b1


def _affine_kernel(x_ref, w_ref, o_ref):
    w = w_ref[0, :_IN_F, :]
    b = w_ref[0, _IN_F:_IN_F + 1, :]
    o_ref[...] = jnp.dot(x_ref[...], w, preferred_element_type=jnp.float32) + b


def _single_affine(x2d, w_fold):
    m = x2d.shape[0]
    tm = 1024
    while tm > 8 and m % tm:
        tm //= 2
    grid = pl.cdiv(m, tm)
    return pl.pallas_call(
        _affine_kernel,
        grid=(grid,),
        in_specs=[
            pl.BlockSpec((tm, _IN_F), lambda i: (i, 0)),
            pl.BlockSpec((1, w_fold.shape[1], _OUT_F), lambda i: (0, 0, 0)),
        ],
        out_specs=pl.BlockSpec((tm, _OUT_F), lambda i: (i, 0)),
        out_shape=jax.ShapeDtypeStruct((m, _OUT_F), x2d.dtype),
        compiler_params=pltpu.CompilerParams(
            dimension_semantics=("parallel",),
            vmem_limit_bytes=64 << 20,
        ),
    )(x2d, w_fold)


def _fused_affine(x0, g0, x1, w_folded, grid):
    m0, mg, m1 = x0.shape[0], g0.shape[0], x1.shape[0]
    bm0, bmg, bm1 = m0 // grid, mg // grid, m1 // grid
    k_rows = w_folded.shape[1]
    return pl.pallas_call(
        _fused_affine_kernel,
        grid=(grid,),
        in_specs=[
            pl.BlockSpec((2, k_rows, _OUT_F), lambda i: (0, 0, 0)),
            pl.BlockSpec((bm0, _IN_F), lambda i: (i, 0)),
            pl.BlockSpec((bmg, _IN_F), lambda i: (i, 0)),
            pl.BlockSpec((bm1, _IN_F), lambda i: (i, 0)),
        ],
        out_specs=[
            pl.BlockSpec((bm0, _OUT_F), lambda i: (i, 0)),
            pl.BlockSpec((bmg, _OUT_F), lambda i: (i, 0)),
            pl.BlockSpec((bm1, _OUT_F), lambda i: (i, 0)),
        ],
        out_shape=[
            jax.ShapeDtypeStruct((m0, _OUT_F), x0.dtype),
            jax.ShapeDtypeStruct((mg, _OUT_F), g0.dtype),
            jax.ShapeDtypeStruct((m1, _OUT_F), x1.dtype),
        ],
        compiler_params=pltpu.CompilerParams(
            dimension_semantics=("parallel",),
            vmem_limit_bytes=64 << 20,
        ),
    )(w_folded, x0, g0, x1)


def kernel(key0_values, key0_grad_positions, key1_values, w_folded, w_block_diag):
    del w_block_diag
    x0 = jnp.asarray(key0_values, jnp.float32).reshape(-1, _IN_F)
    g0 = jnp.asarray(key0_grad_positions, jnp.float32).reshape(-1, _IN_F)
    x1 = jnp.asarray(key1_values, jnp.float32).reshape(-1, _IN_F)

    grid = 0
    for g in (2,):
        if all(m % g == 0 and (m // g) % 8 == 0
               for m in (x0.shape[0], g0.shape[0], x1.shape[0])):
            grid = g
            break
    if grid:
        y0, yg, y1 = _fused_affine(x0, g0, x1, w_folded, grid)
    else:
        y0 = _single_affine(x0, w_folded[0:1])
        yg = _single_affine(g0, w_folded[0:1])
        y1 = _single_affine(x1, w_folded[1:2])

    return {
        "key_0": {
            "values": y0.reshape(*key0_values.shape[:-1], _OUT_F),
            "gradients": {
                "positions": yg.reshape(*key0_grad_positions.shape[:-1], _OUT_F),
            },
        },
        "key_1": {
            "values": y1.reshape(*key1_values.shape[:-1], _OUT_F),
            "gradients": {},
        },
    }
